# baseline (speedup 1.0000x reference)
import jax
import jax.numpy as jnp
from jax.experimental import pallas as pl
from jax.experimental.pallas import tpu as pltpu

NMS_SIZE = 5
MIN_VAL = 1e-05
H = 512
W = 512


def _max5_rows(x):
    z1 = jnp.zeros((1, W), dtype=x.dtype)
    t = jnp.maximum(x, jnp.concatenate([x[1:], z1], axis=0))
    return jnp.maximum(x, jnp.maximum(
        jnp.concatenate([z1, t[:1], t[:-2]], axis=0),
        jnp.concatenate([t[1:], z1], axis=0)))


def _max5_cols(x):
    z1 = jnp.zeros((H, 1), dtype=x.dtype)
    t = jnp.maximum(x, jnp.concatenate([x[:, 1:], z1], axis=1))
    return jnp.maximum(x, jnp.maximum(
        jnp.concatenate([z1, t[:, :1], t[:, :-2]], axis=1),
        jnp.concatenate([t[:, 1:], z1], axis=1)))


HH = H // 2


def _max5_cols_t(x):
    z1 = jnp.zeros((x.shape[0], 1), dtype=x.dtype)
    t = jnp.maximum(x, jnp.concatenate([x[:, 1:], z1], axis=1))
    return jnp.maximum(x, jnp.maximum(
        jnp.concatenate([z1, t[:, :1], t[:, :-2]], axis=1),
        jnp.concatenate([t[:, 1:], z1], axis=1)))


def _nms_body(scale_ref, center_ref, x_hbm, wc_hbm, mask_hbm,
              xbuf, wcbuf, mbuf, insem, wcsem, msem):
    B = x_hbm.shape[0]
    col = jax.lax.broadcasted_iota(jnp.int32, (HH, W), 1).astype(jnp.float32)
    row = jax.lax.broadcasted_iota(jnp.int32, (HH, W), 0).astype(jnp.float32)
    z1 = jnp.zeros((1, W), dtype=jnp.float32)

    def in_copy(b):
        return pltpu.make_async_copy(x_hbm.at[b, 0], xbuf.at[b], insem.at[b])

    def out_copies(b, h):
        r0 = h * HH
        sl = slice(r0, r0 + HH)
        return (pltpu.make_async_copy(wcbuf.at[b, :, sl], wc_hbm.at[b, :, sl],
                                      wcsem.at[b, h]),
                pltpu.make_async_copy(mbuf.at[b, sl], mask_hbm.at[b, sl],
                                      msem.at[b, h]))

    for b in range(B):
        in_copy(b).start()
    for b in range(B):
        in_copy(b).wait()

        x = xbuf[b]
        t = jnp.maximum(x, jnp.concatenate([x[1:], z1], axis=0))
        s = scale_ref[b]
        cx2 = center_ref[2 * b] - (W / 2.0) * s
        cy2 = center_ref[2 * b + 1] + (H / 2.0) * s
        for h in range(2):
            r0 = h * HH
            xc = x[r0:r0 + HH]
            if h == 0:
                tm2 = jnp.concatenate([z1, t[:1], t[:HH - 2]], axis=0)
                tp1 = t[1:HH + 1]
            else:
                tm2 = t[r0 - 2:r0 + HH - 2]
                tp1 = jnp.concatenate([t[r0 + 1:], z1], axis=0)
            v = jnp.maximum(xc, jnp.maximum(tm2, tp1))
            m = _max5_cols_t(v)
            mask = (xc > MIN_VAL) & (xc >= m)
            wcbuf[b, 0, r0:r0 + HH] = jnp.where(mask, col * s + cx2, 0.0)
            wcbuf[b, 1, r0:r0 + HH] = jnp.where(
                mask, (row + float(r0)) * (-s) + cy2, 0.0)
            mbuf[b, r0:r0 + HH] = mask.astype(jnp.int8)
            cwc, cm = out_copies(b, h)
            cwc.start()
            cm.start()

    for b in range(B):
        for h in range(2):
            cwc, cm = out_copies(b, h)
            cwc.wait()
            cm.wait()


def kernel(input_map, bev_scale, bev_center):
    B = input_map.shape[0]
    wc, mask = pl.pallas_call(
        _nms_body,
        in_specs=[
            pl.BlockSpec(memory_space=pltpu.SMEM),
            pl.BlockSpec(memory_space=pltpu.SMEM),
            pl.BlockSpec(memory_space=pltpu.MemorySpace.HBM),
        ],
        out_specs=[
            pl.BlockSpec(memory_space=pltpu.MemorySpace.HBM),
            pl.BlockSpec(memory_space=pltpu.MemorySpace.HBM),
        ],
        out_shape=[
            jax.ShapeDtypeStruct((B, 2, H, W), jnp.float32),
            jax.ShapeDtypeStruct((B, H, W), jnp.int8),
        ],
        scratch_shapes=[
            pltpu.VMEM((4, H, W), jnp.float32),
            pltpu.VMEM((4, 2, H, W), jnp.float32),
            pltpu.VMEM((4, H, W), jnp.int8),
            pltpu.SemaphoreType.DMA((4,)),
            pltpu.SemaphoreType.DMA((4, 2)),
            pltpu.SemaphoreType.DMA((4, 2)),
        ],
    )(bev_scale, bev_center.reshape(-1), input_map)
    return wc, mask.view(jnp.bool_)


# speedup vs baseline: 1.0402x; 1.0402x over previous
import jax
import jax.numpy as jnp
from jax.experimental import pallas as pl
from jax.experimental.pallas import tpu as pltpu

NMS_SIZE = 5
MIN_VAL = 1e-05
H = 512
W = 512


def _max5_rows(x):
    z1 = jnp.zeros((1, W), dtype=x.dtype)
    t = jnp.maximum(x, jnp.concatenate([x[1:], z1], axis=0))
    return jnp.maximum(x, jnp.maximum(
        jnp.concatenate([z1, t[:1], t[:-2]], axis=0),
        jnp.concatenate([t[1:], z1], axis=0)))


def _max5_cols(x):
    z1 = jnp.zeros((H, 1), dtype=x.dtype)
    t = jnp.maximum(x, jnp.concatenate([x[:, 1:], z1], axis=1))
    return jnp.maximum(x, jnp.maximum(
        jnp.concatenate([z1, t[:, :1], t[:, :-2]], axis=1),
        jnp.concatenate([t[:, 1:], z1], axis=1)))


HH = H // 2


def _max5_rows_half(x, t, h):
    z1 = jnp.zeros((1, W), dtype=x.dtype)
    r0 = h * HH
    xc = x[r0:r0 + HH]
    if h == 0:
        tm2 = jnp.concatenate([z1, t[:1], t[:HH - 2]], axis=0)
        tp1 = t[1:HH + 1]
    else:
        tm2 = t[r0 - 2:r0 + HH - 2]
        tp1 = jnp.concatenate([t[r0 + 1:], z1], axis=0)
    return xc, jnp.maximum(xc, jnp.maximum(tm2, tp1))


def _max5_cols_n(x):
    n = x.shape[0]
    z1 = jnp.zeros((n, 1), dtype=x.dtype)
    t = jnp.maximum(x, jnp.concatenate([x[:, 1:], z1], axis=1))
    return jnp.maximum(x, jnp.maximum(
        jnp.concatenate([z1, t[:, :1], t[:, :-2]], axis=1),
        jnp.concatenate([t[:, 1:], z1], axis=1)))


def _nms_body(scale_ref, center_ref, x_hbm, wc_hbm, mask_hbm,
              xbuf, wcbuf, mbuf, insem, wcsem, msem):
    B = x_hbm.shape[0]
    col = jax.lax.broadcasted_iota(jnp.int32, (H, W), 1).astype(jnp.float32)
    row = jax.lax.broadcasted_iota(jnp.int32, (H, W), 0).astype(jnp.float32)

    def in_copy(b):
        return pltpu.make_async_copy(x_hbm.at[b, 0], xbuf.at[b], insem.at[b])

    def out_copies(b):
        return (pltpu.make_async_copy(wcbuf.at[b], wc_hbm.at[b], wcsem.at[b]),
                pltpu.make_async_copy(mbuf.at[b], mask_hbm.at[b], msem.at[b]))

    in0a = pltpu.make_async_copy(x_hbm.at[0, 0, 0:HH], xbuf.at[0, 0:HH],
                                 insem.at[0])
    in0b = pltpu.make_async_copy(x_hbm.at[0, 0, HH:H], xbuf.at[0, HH:H],
                                 wcsem.at[0])
    in0a.start()
    in0b.start()
    for b in range(1, B):
        in_copy(b).start()
    for b in range(B):
        if b == 0:
            in0a.wait()
            in0b.wait()
        else:
            in_copy(b).wait()

        x = xbuf[b]
        s = scale_ref[b]
        cx2 = center_ref[2 * b] - (W / 2.0) * s
        cy2 = center_ref[2 * b + 1] + (H / 2.0) * s
        if b < B - 1:
            t = jnp.maximum(x, jnp.concatenate(
                [x[1:], jnp.zeros((1, W), jnp.float32)], axis=0))
            m = jnp.maximum(x, jnp.maximum(
                jnp.concatenate([jnp.zeros((1, W), jnp.float32), t[:1],
                                 t[:-2]], axis=0),
                jnp.concatenate([t[1:], jnp.zeros((1, W), jnp.float32)],
                                axis=0)))
            m = _max5_cols_n(m)
            mask = (x > MIN_VAL) & (x >= m)
            wcbuf[b, 0] = jnp.where(mask, col * s + cx2, 0.0)
            wcbuf[b, 1] = jnp.where(mask, row * (-s) + cy2, 0.0)
            mbuf[b] = mask.astype(jnp.int8)
            cwc, cm = out_copies(b)
            cwc.start()
            cm.start()
        else:
            t = jnp.maximum(x, jnp.concatenate(
                [x[1:], jnp.zeros((1, W), jnp.float32)], axis=0))
            for h in range(2):
                r0 = h * HH
                xc, v = _max5_rows_half(x, t, h)
                m = _max5_cols_n(v)
                mask = (xc > MIN_VAL) & (xc >= m)
                colh = col[0:HH]
                rowh = row[r0:r0 + HH]
                wcbuf[b, 0, r0:r0 + HH] = jnp.where(mask, colh * s + cx2, 0.0)
                wcbuf[b, 1, r0:r0 + HH] = jnp.where(mask, rowh * (-s) + cy2,
                                                    0.0)
                mbuf[b, r0:r0 + HH] = mask.astype(jnp.int8)
                pltpu.make_async_copy(wcbuf.at[b, 0, r0:r0 + HH],
                                      wc_hbm.at[b, 0, r0:r0 + HH],
                                      wcsem.at[b]).start()
                pltpu.make_async_copy(wcbuf.at[b, 1, r0:r0 + HH],
                                      wc_hbm.at[b, 1, r0:r0 + HH],
                                      msem.at[b]).start()
                pltpu.make_async_copy(mbuf.at[b, r0:r0 + HH],
                                      mask_hbm.at[b, r0:r0 + HH],
                                      insem.at[b]).start()

    for b in range(B - 1):
        cwc, cm = out_copies(b)
        cwc.wait()
        cm.wait()
    bl = B - 1
    for h in range(2):
        r0 = h * HH
        pltpu.make_async_copy(wcbuf.at[bl, 0, r0:r0 + HH],
                              wc_hbm.at[bl, 0, r0:r0 + HH],
                              wcsem.at[bl]).wait()
        pltpu.make_async_copy(wcbuf.at[bl, 1, r0:r0 + HH],
                              wc_hbm.at[bl, 1, r0:r0 + HH],
                              msem.at[bl]).wait()
        pltpu.make_async_copy(mbuf.at[bl, r0:r0 + HH],
                              mask_hbm.at[bl, r0:r0 + HH],
                              insem.at[bl]).wait()


def kernel(input_map, bev_scale, bev_center):
    B = input_map.shape[0]
    wc, mask = pl.pallas_call(
        _nms_body,
        in_specs=[
            pl.BlockSpec(memory_space=pltpu.SMEM),
            pl.BlockSpec(memory_space=pltpu.SMEM),
            pl.BlockSpec(memory_space=pltpu.MemorySpace.HBM),
        ],
        out_specs=[
            pl.BlockSpec(memory_space=pltpu.MemorySpace.HBM),
            pl.BlockSpec(memory_space=pltpu.MemorySpace.HBM),
        ],
        out_shape=[
            jax.ShapeDtypeStruct((B, 2, H, W), jnp.float32),
            jax.ShapeDtypeStruct((B, H, W), jnp.int8),
        ],
        scratch_shapes=[
            pltpu.VMEM((4, H, W), jnp.float32),
            pltpu.VMEM((4, 2, H, W), jnp.float32),
            pltpu.VMEM((4, H, W), jnp.int8),
            pltpu.SemaphoreType.DMA((4,)),
            pltpu.SemaphoreType.DMA((4,)),
            pltpu.SemaphoreType.DMA((4,)),
        ],
    )(bev_scale, bev_center.reshape(-1), input_map)
    return wc, mask.view(jnp.bool_)
